# baseline (speedup 1.0000x reference)
import functools

import jax
import jax.numpy as jnp
from jax import lax
from jax.experimental import pallas as pl
from jax.experimental.pallas import tpu as pltpu
from jax.experimental.pallas import tpu_sc as plsc

F32 = jnp.float32
AF = 128
NF = 16
D1 = 2 * NF
NN = 10000
NE = 320000
NC = 1000
EPS = 1e-5
INV_SQRT_2 = 1.0 / 2.0 ** 0.5

GB = 128
TB = 6400
TBX = 640

_HIGH = jax.lax.Precision.HIGHEST


def _mesh():
    return plsc.VectorSubcoreMesh(core_axis_name="c", subcore_axis_name="s")


_SC_PARAMS = pltpu.CompilerParams(use_tc_tiling_on_sc=False)


def _proj_body(x_ref, w_ref, o_ref):
    o_ref[...] = jnp.dot(x_ref[...], w_ref[...], precision=_HIGH)


def _tc_proj(atom_fea, w1):
    return pl.pallas_call(
        _proj_body,
        out_shape=jax.ShapeDtypeStruct((NN, D1), F32),
    )(atom_fea, w1)


def _ep_body(e_ref, w_ref, o_ref):
    o_ref[...] = jnp.dot(e_ref[...], w_ref[...], precision=_HIGH)


def _tc_ep(edge, w2):
    nb = NE // TB
    return pl.pallas_call(
        _ep_body,
        grid=(nb,),
        in_specs=[
            pl.BlockSpec((TB, NF), lambda i: (i, 0)),
            pl.BlockSpec((NF, D1), lambda i: (0, 0)),
        ],
        out_specs=pl.BlockSpec((TB, D1), lambda i: (i, 0)),
        out_shape=jax.ShapeDtypeStruct((NE, D1), F32),
    )(edge, w2)


def _make_gather_tg():
    @functools.partial(
        pl.kernel,
        out_type=jax.ShapeDtypeStruct((NE, D1), F32),
        mesh=_mesh(),
        compiler_params=_SC_PARAMS,
        scratch_types=[
            pltpu.VMEM((GB, D1), F32),
            pltpu.VMEM((GB, D1), F32),
        ],
    )
    def gather_tg(proj_hbm, idx0_hbm, idx1_hbm, ep_hbm, tg_hbm, p0_v, p1_v):
        def body(idx0_v, idx1_v, ep_v, tg_v):
            pltpu.sync_copy(proj_hbm.at[idx0_v.at[0]], p0_v)
            pltpu.sync_copy(proj_hbm.at[idx1_v.at[0]], p1_v)

            @pl.loop(0, GB)
            def _(r):
                @pl.loop(0, D1, step=16)
                def _(c):
                    slc = (pl.ds(r, 1), pl.ds(c, 16))
                    tg_v.at[slc][...] = (
                        p1_v.at[slc][...] - p0_v.at[slc][...] + ep_v.at[slc][...]
                    )

        pltpu.emit_pipeline(
            body,
            grid=(NE // GB,),
            in_specs=[
                pl.BlockSpec((1, GB), lambda i: (0, i)),
                pl.BlockSpec((1, GB), lambda i: (0, i)),
                pl.BlockSpec((GB, D1), lambda i: (i, 0)),
            ],
            out_specs=[pl.BlockSpec((GB, D1), lambda i: (i, 0))],
            core_axis_name=("c", "s"),
            dimension_semantics=(pltpu.PARALLEL,),
        )(idx0_hbm, idx1_hbm, ep_hbm, tg_hbm)

    return gather_tg


def _make_stats(d, with_cnt):
    out_type = [
        jax.ShapeDtypeStruct((2, NC, d), F32),
        jax.ShapeDtypeStruct((2, NC, d), F32),
    ]
    scratch = [
        pltpu.VMEM((GB, d), F32),
        pltpu.VMEM((125, d), F32),
        pltpu.VMEM_SHARED((NC, d), F32),
        pltpu.VMEM_SHARED((NC, d), F32),
    ]
    if with_cnt:
        out_type.append(jax.ShapeDtypeStruct((2, NC, 16), F32))
        scratch.append(pltpu.VMEM((GB, 16), F32))
        scratch.append(pltpu.VMEM_SHARED((NC, 16), F32))

    @functools.partial(
        pl.kernel, out_type=tuple(out_type), mesh=_mesh(),
        compiler_params=_SC_PARAMS, scratch_types=scratch,
    )
    def stats(*refs):
        if with_cnt:
            (x_hbm, ids_hbm, sum_hbm, sq_hbm, cnt_hbm,
             sq_v, z_v, ssum, ssq, ones_v, scnt) = refs
        else:
            (x_hbm, ids_hbm, sum_hbm, sq_hbm,
             sq_v, z_v, ssum, ssq) = refs
        cid = lax.axis_index("c")
        sid = lax.axis_index("s")

        @pl.when(sid == 0)
        def _():
            @pl.loop(0, 125)
            def _(r):
                @pl.loop(0, d, step=16)
                def _(c):
                    z_v.at[pl.ds(r, 1), pl.ds(c, 16)][...] = jnp.zeros(
                        (1, 16), F32)

            @pl.loop(0, 8)
            def _(k):
                pltpu.sync_copy(z_v, ssum.at[pl.ds(k * 125, 125)])
                pltpu.sync_copy(z_v, ssq.at[pl.ds(k * 125, 125)])
                if with_cnt:
                    pltpu.sync_copy(z_v.at[:, pl.ds(0, 16)],
                                    scnt.at[pl.ds(k * 125, 125)])

        if with_cnt:
            @pl.loop(0, GB)
            def _(r):
                ones_v.at[pl.ds(r, 1), pl.ds(0, 16)][...] = jnp.ones(
                    (1, 16), F32)

        plsc.subcore_barrier()

        def body(x_v, ids_v):
            @pl.loop(0, GB)
            def _(r):
                @pl.loop(0, d, step=16)
                def _(c):
                    slc = (pl.ds(r, 1), pl.ds(c, 16))
                    v = x_v.at[slc][...]
                    sq_v.at[slc][...] = v * v

            pltpu.sync_copy(x_v, ssum.at[ids_v.at[0]], add=True)
            pltpu.sync_copy(sq_v, ssq.at[ids_v.at[0]], add=True)
            if with_cnt:
                pltpu.sync_copy(ones_v, scnt.at[ids_v.at[0]], add=True)

        pltpu.emit_pipeline(
            body,
            grid=(NE // GB,),
            in_specs=[
                pl.BlockSpec((GB, d), lambda i: (i, 0)),
                pl.BlockSpec((1, GB), lambda i: (0, i)),
            ],
            out_specs=[],
            core_axis_name=("c", "s"),
            dimension_semantics=(pltpu.PARALLEL,),
        )(x_hbm, ids_hbm)

        plsc.subcore_barrier()

        @pl.when(sid == 0)
        def _():
            pltpu.sync_copy(ssum, sum_hbm.at[cid])
            pltpu.sync_copy(ssq, sq_hbm.at[cid])
            if with_cnt:
                pltpu.sync_copy(scnt, cnt_hbm.at[cid])

    return stats


def _fin_body(sum_ref, sq_ref, cnt_ref, g_ref, bt_ref, a_ref, b_ref):
    s = sum_ref[0] + sum_ref[1]
    q = sq_ref[0] + sq_ref[1]
    n = jnp.maximum(cnt_ref[0, :, 0:1] + cnt_ref[1, :, 0:1], 1.0)
    mean = s / n
    var = jnp.maximum(q / n - mean * mean, 0.0)
    a = g_ref[...] * lax.rsqrt(var + EPS)
    a_ref[...] = a
    b_ref[...] = bt_ref[...] - mean * a


def _tc_finalize(d, ssum, ssq, cnt, gamma, beta):
    return pl.pallas_call(
        _fin_body,
        out_shape=(
            jax.ShapeDtypeStruct((NC, d), F32),
            jax.ShapeDtypeStruct((NC, d), F32),
        ),
    )(ssum, ssq, cnt, gamma.reshape(1, d), beta.reshape(1, d))


def _expand_tables(ids, a_ref, b_ref, d):
    clo = jnp.min(ids)
    chi = jnp.max(ids)
    zero = jnp.zeros((ids.shape[0], d), F32)

    def it(c, ab):
        a_acc, b_acc = ab
        m = (ids == c).astype(F32)
        a_acc = a_acc + m * a_ref[pl.ds(c, 1), :]
        b_acc = b_acc + m * b_ref[pl.ds(c, 1), :]
        return (a_acc, b_acc)

    return lax.fori_loop(clo, chi + 1, it, (zero, zero))


def _gate_body(tg_ref, ids_ref, a_ref, b_ref, wm_ref, ns_ref):
    ids = ids_ref[...]
    a_pe, b_pe = _expand_tables(ids, a_ref, b_ref, D1)
    tgn = tg_ref[...] * a_pe + b_pe
    filt = jnp.tanh(jnp.dot(tgn, wm_ref[...], precision=_HIGH))
    ns_ref[...] = (jax.nn.relu(tgn) * filt)[:, :NF]


def _tc_gate(tg, ids_col, ta1, tb1, wm_pad):
    nb = NE // TBX
    return pl.pallas_call(
        _gate_body,
        grid=(nb,),
        in_specs=[
            pl.BlockSpec((TBX, D1), lambda i: (i, 0)),
            pl.BlockSpec((TBX, 1), lambda i: (i, 0)),
            pl.BlockSpec((NC, D1), lambda i: (0, 0)),
            pl.BlockSpec((NC, D1), lambda i: (0, 0)),
            pl.BlockSpec((D1, 1), lambda i: (0, 0)),
        ],
        out_specs=pl.BlockSpec((TBX, NF), lambda i: (i, 0)),
        out_shape=jax.ShapeDtypeStruct((NE, NF), F32),
    )(tg, ids_col, ta1, tb1, wm_pad)


def _final_body(ns_ref, ids_ref, a_ref, b_ref, e_ref,
                w1a_ref, b1a_ref, w2a_ref, b2a_ref,
                w1b_ref, b1b_ref, w2b_ref, b2b_ref, o_ref):
    ids = ids_ref[...]
    a_pe, b_pe = _expand_tables(ids, a_ref, b_ref, NF)
    x = ns_ref[...] * a_pe + b_pe
    h = jnp.dot(jax.nn.relu(jnp.dot(x, w1a_ref[...], precision=_HIGH)
                            + b1a_ref[...]),
                w2a_ref[...], precision=_HIGH) + b2a_ref[...]
    x = x + h
    h = jnp.dot(jax.nn.relu(jnp.dot(x, w1b_ref[...], precision=_HIGH)
                            + b1b_ref[...]),
                w2b_ref[...], precision=_HIGH) + b2b_ref[...]
    x = x + h
    o_ref[...] = INV_SQRT_2 * jax.nn.relu(e_ref[...] + x)


def _tc_final(ns, ids_col, ta2, tb2, edge, rw):
    nb = NE // TBX
    mid = NF // 2
    eb = lambda i: (i, 0)
    wb = lambda i: (0, 0)
    return pl.pallas_call(
        _final_body,
        grid=(nb,),
        in_specs=[
            pl.BlockSpec((TBX, NF), eb),
            pl.BlockSpec((TBX, 1), eb),
            pl.BlockSpec((NC, NF), wb),
            pl.BlockSpec((NC, NF), wb),
            pl.BlockSpec((TBX, NF), eb),
            pl.BlockSpec((NF, mid), wb),
            pl.BlockSpec((1, mid), wb),
            pl.BlockSpec((mid, NF), wb),
            pl.BlockSpec((1, NF), wb),
            pl.BlockSpec((NF, mid), wb),
            pl.BlockSpec((1, mid), wb),
            pl.BlockSpec((mid, NF), wb),
            pl.BlockSpec((1, NF), wb),
        ],
        out_specs=pl.BlockSpec((TBX, NF), eb),
        out_shape=jax.ShapeDtypeStruct((NE, NF), F32),
    )(ns, ids_col, ta2, tb2, edge, *rw)


def kernel(atom_fea, edge, crystal_atom_idx, crystal_edge_idx, nbr_fea_idx,
           rbf, W_full, W_mask, res_W1a, res_b1a, res_W2a, res_b2a,
           res_W1b, res_b1b, res_W2b, res_b2b, gamma1, beta1, gamma2, beta2):
    ids_i32 = crystal_edge_idx.astype(jnp.int32)
    ids_row = ids_i32.reshape(1, NE)
    ids_col = ids_i32.reshape(NE, 1)
    nbr_t = nbr_fea_idx.astype(jnp.int32).T
    idx0 = nbr_t[0].reshape(1, NE)
    idx1 = nbr_t[1].reshape(1, NE)
    w1 = W_full[:AF, :]
    w2 = W_full[AF:, :]
    wm_pad = jnp.concatenate([jnp.zeros((NF, 1), F32), W_mask], axis=0)

    proj = _tc_proj(atom_fea, w1)
    ep = _tc_ep(edge, w2)
    tg = _make_gather_tg()(proj, idx0, idx1, ep)

    s1, q1, cnt = _make_stats(D1, True)(tg, ids_row)
    ta1, tb1 = _tc_finalize(D1, s1, q1, cnt, gamma1, beta1)
    ns = _tc_gate(tg, ids_col, ta1, tb1, wm_pad)

    s2, q2 = _make_stats(NF, False)(ns, ids_row)
    ta2, tb2 = _tc_finalize(NF, s2, q2, cnt, gamma2, beta2)

    rw = (res_W1a, res_b1a.reshape(1, -1), res_W2a, res_b2a.reshape(1, -1),
          res_W1b, res_b1b.reshape(1, -1), res_W2b, res_b2b.reshape(1, -1))
    return _tc_final(ns, ids_col, ta2, tb2, edge, rw)


# speedup vs baseline: 1.7866x; 1.7866x over previous
import functools

import jax
import jax.numpy as jnp
from jax import lax
from jax.experimental import pallas as pl
from jax.experimental.pallas import tpu as pltpu
from jax.experimental.pallas import tpu_sc as plsc

F32 = jnp.float32
AF = 128
NF = 16
D1 = 2 * NF
NN = 10000
NE = 320000
NC = 1000
EPS = 1e-5
INV_SQRT_2 = 1.0 / 2.0 ** 0.5

GB = 128
TB = 6400
TBX = 1280
EW = 128

_HIGH = jax.lax.Precision.HIGHEST


def _mesh():
    return plsc.VectorSubcoreMesh(core_axis_name="c", subcore_axis_name="s")


_SC_PARAMS = pltpu.CompilerParams(use_tc_tiling_on_sc=False)


def _proj_body(x_ref, w_ref, o_ref):
    o_ref[...] = jnp.dot(x_ref[...], w_ref[...], precision=_HIGH)


def _tc_proj(atom_fea, w1):
    return pl.pallas_call(
        _proj_body,
        out_shape=jax.ShapeDtypeStruct((NN, D1), F32),
    )(atom_fea, w1)


def _ep_body(e_ref, w_ref, o_ref):
    o_ref[...] = jnp.dot(e_ref[...], w_ref[...])


def _tc_ep(edge, w2):
    nb = NE // TB
    return pl.pallas_call(
        _ep_body,
        grid=(nb,),
        in_specs=[
            pl.BlockSpec((TB, NF), lambda i: (i, 0)),
            pl.BlockSpec((NF, D1), lambda i: (0, 0)),
        ],
        out_specs=pl.BlockSpec((TB, D1), lambda i: (i, 0)),
        out_shape=jax.ShapeDtypeStruct((NE, D1), F32),
    )(edge, w2)


def _make_gather_tg():
    @functools.partial(
        pl.kernel,
        out_type=jax.ShapeDtypeStruct((NE, D1), F32),
        mesh=_mesh(),
        compiler_params=_SC_PARAMS,
        scratch_types=[
            pltpu.VMEM((GB, D1), F32),
            pltpu.VMEM((GB, D1), F32),
        ],
    )
    def gather_tg(proj_hbm, idx0_hbm, idx1_hbm, ep_hbm, tg_hbm, p0_v, p1_v):
        def body(idx0_v, idx1_v, ep_v, tg_v):
            pltpu.sync_copy(proj_hbm.at[idx0_v.at[0]], p0_v)
            pltpu.sync_copy(proj_hbm.at[idx1_v.at[0]], p1_v)

            @pl.loop(0, GB)
            def _(r):
                @pl.loop(0, D1, step=16)
                def _(c):
                    slc = (pl.ds(r, 1), pl.ds(c, 16))
                    tg_v.at[slc][...] = (
                        p1_v.at[slc][...] - p0_v.at[slc][...] + ep_v.at[slc][...]
                    )

        pltpu.emit_pipeline(
            body,
            grid=(NE // GB,),
            in_specs=[
                pl.BlockSpec((1, GB), lambda i: (0, i)),
                pl.BlockSpec((1, GB), lambda i: (0, i)),
                pl.BlockSpec((GB, D1), lambda i: (i, 0)),
            ],
            out_specs=[pl.BlockSpec((GB, D1), lambda i: (i, 0))],
            core_axis_name=("c", "s"),
            dimension_semantics=(pltpu.PARALLEL,),
        )(idx0_hbm, idx1_hbm, ep_hbm, tg_hbm)

    return gather_tg


def _make_stats(d, with_cnt):
    out_type = [
        jax.ShapeDtypeStruct((2, NC, d), F32),
        jax.ShapeDtypeStruct((2, NC, d), F32),
    ]
    scratch = [
        pltpu.VMEM((GB, d), F32),
        pltpu.VMEM((125, d), F32),
        pltpu.VMEM_SHARED((NC, d), F32),
        pltpu.VMEM_SHARED((NC, d), F32),
    ]
    if with_cnt:
        out_type.append(jax.ShapeDtypeStruct((2, NC, 16), F32))
        scratch.append(pltpu.VMEM((GB, 16), F32))
        scratch.append(pltpu.VMEM_SHARED((NC, 16), F32))

    @functools.partial(
        pl.kernel, out_type=tuple(out_type), mesh=_mesh(),
        compiler_params=_SC_PARAMS, scratch_types=scratch,
    )
    def stats(*refs):
        if with_cnt:
            (x_hbm, ids_hbm, sum_hbm, sq_hbm, cnt_hbm,
             sq_v, z_v, ssum, ssq, ones_v, scnt) = refs
        else:
            (x_hbm, ids_hbm, sum_hbm, sq_hbm,
             sq_v, z_v, ssum, ssq) = refs
        cid = lax.axis_index("c")
        sid = lax.axis_index("s")

        @pl.when(sid == 0)
        def _():
            @pl.loop(0, 125)
            def _(r):
                @pl.loop(0, d, step=16)
                def _(c):
                    z_v.at[pl.ds(r, 1), pl.ds(c, 16)][...] = jnp.zeros(
                        (1, 16), F32)

            @pl.loop(0, 8)
            def _(k):
                pltpu.sync_copy(z_v, ssum.at[pl.ds(k * 125, 125)])
                pltpu.sync_copy(z_v, ssq.at[pl.ds(k * 125, 125)])
                if with_cnt:
                    pltpu.sync_copy(z_v.at[:, pl.ds(0, 16)],
                                    scnt.at[pl.ds(k * 125, 125)])

        if with_cnt:
            @pl.loop(0, GB)
            def _(r):
                ones_v.at[pl.ds(r, 1), pl.ds(0, 16)][...] = jnp.ones(
                    (1, 16), F32)

        plsc.subcore_barrier()

        def body(x_v, ids_v):
            @pl.loop(0, GB)
            def _(r):
                @pl.loop(0, d, step=16)
                def _(c):
                    slc = (pl.ds(r, 1), pl.ds(c, 16))
                    v = x_v.at[slc][...]
                    sq_v.at[slc][...] = v * v

            pltpu.sync_copy(x_v, ssum.at[ids_v.at[0]], add=True)
            pltpu.sync_copy(sq_v, ssq.at[ids_v.at[0]], add=True)
            if with_cnt:
                pltpu.sync_copy(ones_v, scnt.at[ids_v.at[0]], add=True)

        pltpu.emit_pipeline(
            body,
            grid=(NE // GB,),
            in_specs=[
                pl.BlockSpec((GB, d), lambda i: (i, 0)),
                pl.BlockSpec((1, GB), lambda i: (0, i)),
            ],
            out_specs=[],
            core_axis_name=("c", "s"),
            dimension_semantics=(pltpu.PARALLEL,),
        )(x_hbm, ids_hbm)

        plsc.subcore_barrier()

        @pl.when(sid == 0)
        def _():
            pltpu.sync_copy(ssum, sum_hbm.at[cid])
            pltpu.sync_copy(ssq, sq_hbm.at[cid])
            if with_cnt:
                pltpu.sync_copy(scnt, cnt_hbm.at[cid])

    return stats


def _fin_body(sum_ref, sq_ref, cnt_ref, g_ref, bt_ref,
              a_ref, b_ref, ah_ref, al_ref, bh_ref, bl_ref):
    s = sum_ref[0] + sum_ref[1]
    q = sq_ref[0] + sq_ref[1]
    n = jnp.maximum(cnt_ref[0, :, 0:1] + cnt_ref[1, :, 0:1], 1.0)
    mean = s / n
    var = jnp.maximum(q / n - mean * mean, 0.0)
    a = g_ref[...] * lax.rsqrt(var + EPS)
    b = bt_ref[...] - mean * a
    a_ref[...] = a
    b_ref[...] = b
    ah = a.astype(jnp.bfloat16)
    bh = b.astype(jnp.bfloat16)
    ah_ref[...] = ah
    bh_ref[...] = bh
    al_ref[...] = (a - ah.astype(F32)).astype(jnp.bfloat16)
    bl_ref[...] = (b - bh.astype(F32)).astype(jnp.bfloat16)


def _tc_finalize(d, ssum, ssq, cnt, gamma, beta):
    bf = jnp.bfloat16
    return pl.pallas_call(
        _fin_body,
        out_shape=(
            jax.ShapeDtypeStruct((NC, d), F32),
            jax.ShapeDtypeStruct((NC, d), F32),
            jax.ShapeDtypeStruct((NC, d), bf),
            jax.ShapeDtypeStruct((NC, d), bf),
            jax.ShapeDtypeStruct((NC, d), bf),
            jax.ShapeDtypeStruct((NC, d), bf),
        ),
    )(ssum, ssq, cnt, gamma.reshape(1, d), beta.reshape(1, d))


def _expand_tables(ids, ah_ref, al_ref, bh_ref, bl_ref, a_ref, b_ref, d):
    n = ids.shape[0]
    clo = jnp.min(ids)
    chi = jnp.max(ids)
    base = pl.multiple_of(jnp.minimum((clo // 8) * 8, NC - EW), 8)
    lid = ids - base
    iota = lax.broadcasted_iota(jnp.int32, (1, EW), 1)
    oh = (lid == iota).astype(jnp.bfloat16)
    wah = ah_ref[pl.ds(base, EW), :]
    wal = al_ref[pl.ds(base, EW), :]
    wbh = bh_ref[pl.ds(base, EW), :]
    wbl = bl_ref[pl.ds(base, EW), :]
    a_pe = (jnp.dot(oh, wah, preferred_element_type=F32)
            + jnp.dot(oh, wal, preferred_element_type=F32))
    b_pe = (jnp.dot(oh, wbh, preferred_element_type=F32)
            + jnp.dot(oh, wbl, preferred_element_type=F32))

    def it(c, ab):
        a_acc, b_acc = ab
        m = (ids == c).astype(F32)
        a_acc = a_acc + m * a_ref[pl.ds(c, 1), :]
        b_acc = b_acc + m * b_ref[pl.ds(c, 1), :]
        return (a_acc, b_acc)

    return lax.fori_loop(base + EW, chi + 1, it, (a_pe, b_pe))


def _gate_body(tg_ref, ids_ref, ah_ref, al_ref, bh_ref, bl_ref,
               a_ref, b_ref, wm_ref, ns_ref):
    ids = ids_ref[...]
    a_pe, b_pe = _expand_tables(ids, ah_ref, al_ref, bh_ref, bl_ref,
                                a_ref, b_ref, D1)
    tgn = tg_ref[...] * a_pe + b_pe
    filt = jnp.tanh(jnp.dot(tgn, wm_ref[...]))
    ns_ref[...] = (jax.nn.relu(tgn) * filt)[:, :NF]


def _tc_gate(tg, ids_col, tabs1, wm_pad):
    nb = NE // TBX
    eb = lambda i: (i, 0)
    wb = lambda i: (0, 0)
    return pl.pallas_call(
        _gate_body,
        grid=(nb,),
        in_specs=[
            pl.BlockSpec((TBX, D1), eb),
            pl.BlockSpec((TBX, 1), eb),
            pl.BlockSpec((NC, D1), wb),
            pl.BlockSpec((NC, D1), wb),
            pl.BlockSpec((NC, D1), wb),
            pl.BlockSpec((NC, D1), wb),
            pl.BlockSpec((NC, D1), wb),
            pl.BlockSpec((NC, D1), wb),
            pl.BlockSpec((D1, 1), wb),
        ],
        out_specs=pl.BlockSpec((TBX, NF), eb),
        out_shape=jax.ShapeDtypeStruct((NE, NF), F32),
    )(tg, ids_col, tabs1[2], tabs1[3], tabs1[4], tabs1[5],
      tabs1[0], tabs1[1], wm_pad)


def _final_body(ns_ref, ids_ref, ah_ref, al_ref, bh_ref, bl_ref,
                a_ref, b_ref, e_ref,
                w1a_ref, b1a_ref, w2a_ref, b2a_ref,
                w1b_ref, b1b_ref, w2b_ref, b2b_ref, o_ref):
    ids = ids_ref[...]
    a_pe, b_pe = _expand_tables(ids, ah_ref, al_ref, bh_ref, bl_ref,
                                a_ref, b_ref, NF)
    x = ns_ref[...] * a_pe + b_pe
    h = jnp.dot(jax.nn.relu(jnp.dot(x, w1a_ref[...]) + b1a_ref[...]),
                w2a_ref[...]) + b2a_ref[...]
    x = x + h
    h = jnp.dot(jax.nn.relu(jnp.dot(x, w1b_ref[...]) + b1b_ref[...]),
                w2b_ref[...]) + b2b_ref[...]
    x = x + h
    o_ref[...] = INV_SQRT_2 * jax.nn.relu(e_ref[...] + x)


def _tc_final(ns, ids_col, tabs2, edge, rw):
    nb = NE // TBX
    mid = NF // 2
    eb = lambda i: (i, 0)
    wb = lambda i: (0, 0)
    return pl.pallas_call(
        _final_body,
        grid=(nb,),
        in_specs=[
            pl.BlockSpec((TBX, NF), eb),
            pl.BlockSpec((TBX, 1), eb),
            pl.BlockSpec((NC, NF), wb),
            pl.BlockSpec((NC, NF), wb),
            pl.BlockSpec((NC, NF), wb),
            pl.BlockSpec((NC, NF), wb),
            pl.BlockSpec((NC, NF), wb),
            pl.BlockSpec((NC, NF), wb),
            pl.BlockSpec((TBX, NF), eb),
            pl.BlockSpec((NF, mid), wb),
            pl.BlockSpec((1, mid), wb),
            pl.BlockSpec((mid, NF), wb),
            pl.BlockSpec((1, NF), wb),
            pl.BlockSpec((NF, mid), wb),
            pl.BlockSpec((1, mid), wb),
            pl.BlockSpec((mid, NF), wb),
            pl.BlockSpec((1, NF), wb),
        ],
        out_specs=pl.BlockSpec((TBX, NF), eb),
        out_shape=jax.ShapeDtypeStruct((NE, NF), F32),
    )(ns, ids_col, tabs2[2], tabs2[3], tabs2[4], tabs2[5],
      tabs2[0], tabs2[1], edge, *rw)


def kernel(atom_fea, edge, crystal_atom_idx, crystal_edge_idx, nbr_fea_idx,
           rbf, W_full, W_mask, res_W1a, res_b1a, res_W2a, res_b2a,
           res_W1b, res_b1b, res_W2b, res_b2b, gamma1, beta1, gamma2, beta2):
    ids_i32 = crystal_edge_idx.astype(jnp.int32)
    ids_row = ids_i32.reshape(1, NE)
    ids_col = ids_i32.reshape(NE, 1)
    nbr_t = nbr_fea_idx.astype(jnp.int32).T
    idx0 = nbr_t[0].reshape(1, NE)
    idx1 = nbr_t[1].reshape(1, NE)
    w1 = W_full[:AF, :]
    w2 = W_full[AF:, :]
    wm_pad = jnp.concatenate([jnp.zeros((NF, 1), F32), W_mask], axis=0)

    proj = _tc_proj(atom_fea, w1)
    ep = _tc_ep(edge, w2)
    tg = _make_gather_tg()(proj, idx0, idx1, ep)

    s1, q1, cnt = _make_stats(D1, True)(tg, ids_row)
    tabs1 = _tc_finalize(D1, s1, q1, cnt, gamma1, beta1)
    ns = _tc_gate(tg, ids_col, tabs1, wm_pad)

    s2, q2 = _make_stats(NF, False)(ns, ids_row)
    tabs2 = _tc_finalize(NF, s2, q2, cnt, gamma2, beta2)

    rw = (res_W1a, res_b1a.reshape(1, -1), res_W2a, res_b2a.reshape(1, -1),
          res_W1b, res_b1b.reshape(1, -1), res_W2b, res_b2b.reshape(1, -1))
    return _tc_final(ns, ids_col, tabs2, edge, rw)


# speedup vs baseline: 1.9192x; 1.0743x over previous
import functools

import jax
import jax.numpy as jnp
from jax import lax
from jax.experimental import pallas as pl
from jax.experimental.pallas import tpu as pltpu
from jax.experimental.pallas import tpu_sc as plsc

F32 = jnp.float32
AF = 128
NF = 16
D1 = 2 * NF
NN = 10000
NE = 320000
NC = 1000
EPS = 1e-5
INV_SQRT_2 = 1.0 / 2.0 ** 0.5

GB = 128
TB = 6400
TBX = 1280
EW = 128

_HIGH = jax.lax.Precision.HIGHEST


def _mesh():
    return plsc.VectorSubcoreMesh(core_axis_name="c", subcore_axis_name="s")


_SC_PARAMS = pltpu.CompilerParams(use_tc_tiling_on_sc=False)


def _proj_body(x_ref, w_ref, o_ref):
    o_ref[...] = jnp.dot(x_ref[...], w_ref[...], precision=_HIGH)


def _tc_proj(atom_fea, w1):
    return pl.pallas_call(
        _proj_body,
        out_shape=jax.ShapeDtypeStruct((NN, D1), F32),
    )(atom_fea, w1)


def _ep_body(e_ref, w_ref, o_ref):
    o_ref[...] = jnp.dot(e_ref[...], w_ref[...])


def _tc_ep(edge, w2):
    nb = NE // TB
    return pl.pallas_call(
        _ep_body,
        grid=(nb,),
        in_specs=[
            pl.BlockSpec((TB, NF), lambda i: (i, 0)),
            pl.BlockSpec((NF, D1), lambda i: (0, 0)),
        ],
        out_specs=pl.BlockSpec((TB, D1), lambda i: (i, 0)),
        out_shape=jax.ShapeDtypeStruct((NE, D1), F32),
    )(edge, w2)


SB = 512
SUB = SB // GB


def _make_gather_tg_stats():
    @functools.partial(
        pl.kernel,
        out_type=(
            jax.ShapeDtypeStruct((NE, D1), F32),
            jax.ShapeDtypeStruct((2, NC, D1), F32),
            jax.ShapeDtypeStruct((2, NC, D1), F32),
            jax.ShapeDtypeStruct((2, NC, 16), F32),
        ),
        mesh=_mesh(),
        compiler_params=_SC_PARAMS,
        scratch_types=[
            pltpu.VMEM((SB, D1), F32),
            pltpu.VMEM((SB, D1), F32),
            pltpu.VMEM((SB, D1), F32),
            pltpu.VMEM((GB, 16), F32),
            pltpu.VMEM((125, D1), F32),
            pltpu.VMEM_SHARED((NC, D1), F32),
            pltpu.VMEM_SHARED((NC, D1), F32),
            pltpu.VMEM_SHARED((NC, 16), F32),
            pltpu.SemaphoreType.DMA,
            pltpu.SemaphoreType.DMA,
        ],
    )
    def gather_tg(proj_hbm, idx0_hbm, idx1_hbm, ids_hbm, ep_hbm,
                  tg_hbm, sum_hbm, sq_hbm, cnt_hbm,
                  p0_v, p1_v, sq_v, ones_v, z_v, ssum, ssq, scnt,
                  gsem, wsem):
        cid = lax.axis_index("c")
        sid = lax.axis_index("s")

        @pl.when(sid == 0)
        def _():
            @pl.loop(0, 125)
            def _(r):
                @pl.loop(0, D1, step=16)
                def _(c):
                    z_v.at[pl.ds(r, 1), pl.ds(c, 16)][...] = jnp.zeros(
                        (1, 16), F32)

            @pl.loop(0, 8)
            def _(k):
                pltpu.sync_copy(z_v, ssum.at[pl.ds(k * 125, 125)])
                pltpu.sync_copy(z_v, ssq.at[pl.ds(k * 125, 125)])
                pltpu.sync_copy(z_v.at[:, pl.ds(0, 16)],
                                scnt.at[pl.ds(k * 125, 125)])

        @pl.loop(0, GB)
        def _(r):
            ones_v.at[pl.ds(r, 1), pl.ds(0, 16)][...] = jnp.ones((1, 16), F32)

        plsc.subcore_barrier()

        def body(idx0_v, idx1_v, ids_v, ep_v, tg_v):
            cps = []
            for k in range(SUB):
                dst = pl.ds(k * GB, GB)
                cps.append(pltpu.async_copy(
                    proj_hbm.at[idx0_v.at[k]], p0_v.at[dst], gsem))
                cps.append(pltpu.async_copy(
                    proj_hbm.at[idx1_v.at[k]], p1_v.at[dst], gsem))
            for cp in cps:
                cp.wait()

            @pl.loop(0, SB)
            def _(r):
                @pl.loop(0, D1, step=16)
                def _(c):
                    slc = (pl.ds(r, 1), pl.ds(c, 16))
                    v = (p1_v.at[slc][...] - p0_v.at[slc][...]
                         + ep_v.at[slc][...])
                    tg_v.at[slc][...] = v
                    sq_v.at[slc][...] = v * v

            cps = []
            for k in range(SUB):
                src = pl.ds(k * GB, GB)
                cps.append(pltpu.async_copy(
                    tg_v.at[src], ssum.at[ids_v.at[k]], wsem, add=True))
                cps.append(pltpu.async_copy(
                    sq_v.at[src], ssq.at[ids_v.at[k]], wsem, add=True))
                cps.append(pltpu.async_copy(
                    ones_v, scnt.at[ids_v.at[k]], wsem, add=True))
            for cp in cps:
                cp.wait()

        pltpu.emit_pipeline(
            body,
            grid=(NE // SB,),
            in_specs=[
                pl.BlockSpec((SUB, GB), lambda i: (i, 0)),
                pl.BlockSpec((SUB, GB), lambda i: (i, 0)),
                pl.BlockSpec((SUB, GB), lambda i: (i, 0)),
                pl.BlockSpec((SB, D1), lambda i: (i, 0)),
            ],
            out_specs=[pl.BlockSpec((SB, D1), lambda i: (i, 0))],
            core_axis_name=("c", "s"),
            dimension_semantics=(pltpu.PARALLEL,),
        )(idx0_hbm, idx1_hbm, ids_hbm, ep_hbm, tg_hbm)

        plsc.subcore_barrier()

        @pl.when(sid == 0)
        def _():
            pltpu.sync_copy(ssum, sum_hbm.at[cid])
            pltpu.sync_copy(ssq, sq_hbm.at[cid])
            pltpu.sync_copy(scnt, cnt_hbm.at[cid])

    return gather_tg


def _make_stats(d):
    @functools.partial(
        pl.kernel,
        out_type=(
            jax.ShapeDtypeStruct((2, NC, d), F32),
            jax.ShapeDtypeStruct((2, NC, d), F32),
        ),
        mesh=_mesh(),
        compiler_params=_SC_PARAMS,
        scratch_types=[
            pltpu.VMEM((SB, d), F32),
            pltpu.VMEM((125, d), F32),
            pltpu.VMEM_SHARED((NC, d), F32),
            pltpu.VMEM_SHARED((NC, d), F32),
            pltpu.SemaphoreType.DMA,
        ],
    )
    def stats(x_hbm, ids_hbm, sum_hbm, sq_hbm, sq_v, z_v, ssum, ssq, wsem):
        cid = lax.axis_index("c")
        sid = lax.axis_index("s")

        @pl.when(sid == 0)
        def _():
            @pl.loop(0, 125)
            def _(r):
                @pl.loop(0, d, step=16)
                def _(c):
                    z_v.at[pl.ds(r, 1), pl.ds(c, 16)][...] = jnp.zeros(
                        (1, 16), F32)

            @pl.loop(0, 8)
            def _(k):
                pltpu.sync_copy(z_v, ssum.at[pl.ds(k * 125, 125)])
                pltpu.sync_copy(z_v, ssq.at[pl.ds(k * 125, 125)])

        plsc.subcore_barrier()

        def body(x_v, ids_v):
            @pl.loop(0, SB)
            def _(r):
                @pl.loop(0, d, step=16)
                def _(c):
                    slc = (pl.ds(r, 1), pl.ds(c, 16))
                    v = x_v.at[slc][...]
                    sq_v.at[slc][...] = v * v

            cps = []
            for k in range(SUB):
                src = pl.ds(k * GB, GB)
                cps.append(pltpu.async_copy(
                    x_v.at[src], ssum.at[ids_v.at[k]], wsem, add=True))
                cps.append(pltpu.async_copy(
                    sq_v.at[src], ssq.at[ids_v.at[k]], wsem, add=True))
            for cp in cps:
                cp.wait()

        pltpu.emit_pipeline(
            body,
            grid=(NE // SB,),
            in_specs=[
                pl.BlockSpec((SB, d), lambda i: (i, 0)),
                pl.BlockSpec((SUB, GB), lambda i: (i, 0)),
            ],
            out_specs=[],
            core_axis_name=("c", "s"),
            dimension_semantics=(pltpu.PARALLEL,),
        )(x_hbm, ids_hbm)

        plsc.subcore_barrier()

        @pl.when(sid == 0)
        def _():
            pltpu.sync_copy(ssum, sum_hbm.at[cid])
            pltpu.sync_copy(ssq, sq_hbm.at[cid])

    return stats


def _fin_body(sum_ref, sq_ref, cnt_ref, g_ref, bt_ref,
              a_ref, b_ref, ah_ref, al_ref, bh_ref, bl_ref):
    s = sum_ref[0] + sum_ref[1]
    q = sq_ref[0] + sq_ref[1]
    n = jnp.maximum(cnt_ref[0, :, 0:1] + cnt_ref[1, :, 0:1], 1.0)
    mean = s / n
    var = jnp.maximum(q / n - mean * mean, 0.0)
    a = g_ref[...] * lax.rsqrt(var + EPS)
    b = bt_ref[...] - mean * a
    a_ref[...] = a
    b_ref[...] = b
    ah = a.astype(jnp.bfloat16)
    bh = b.astype(jnp.bfloat16)
    ah_ref[...] = ah
    bh_ref[...] = bh
    al_ref[...] = (a - ah.astype(F32)).astype(jnp.bfloat16)
    bl_ref[...] = (b - bh.astype(F32)).astype(jnp.bfloat16)


def _tc_finalize(d, ssum, ssq, cnt, gamma, beta):
    bf = jnp.bfloat16
    return pl.pallas_call(
        _fin_body,
        out_shape=(
            jax.ShapeDtypeStruct((NC, d), F32),
            jax.ShapeDtypeStruct((NC, d), F32),
            jax.ShapeDtypeStruct((NC, d), bf),
            jax.ShapeDtypeStruct((NC, d), bf),
            jax.ShapeDtypeStruct((NC, d), bf),
            jax.ShapeDtypeStruct((NC, d), bf),
        ),
    )(ssum, ssq, cnt, gamma.reshape(1, d), beta.reshape(1, d))


def _expand_tables(ids, ah_ref, al_ref, bh_ref, bl_ref, a_ref, b_ref, d):
    n = ids.shape[0]
    clo = jnp.min(ids)
    chi = jnp.max(ids)
    base = pl.multiple_of(jnp.minimum((clo // 8) * 8, NC - EW), 8)
    lid = ids - base
    iota = lax.broadcasted_iota(jnp.int32, (1, EW), 1)
    oh = (lid == iota).astype(jnp.bfloat16)
    wah = ah_ref[pl.ds(base, EW), :]
    wal = al_ref[pl.ds(base, EW), :]
    wbh = bh_ref[pl.ds(base, EW), :]
    wbl = bl_ref[pl.ds(base, EW), :]
    a_pe = (jnp.dot(oh, wah, preferred_element_type=F32)
            + jnp.dot(oh, wal, preferred_element_type=F32))
    b_pe = (jnp.dot(oh, wbh, preferred_element_type=F32)
            + jnp.dot(oh, wbl, preferred_element_type=F32))

    def it(c, ab):
        a_acc, b_acc = ab
        m = (ids == c).astype(F32)
        a_acc = a_acc + m * a_ref[pl.ds(c, 1), :]
        b_acc = b_acc + m * b_ref[pl.ds(c, 1), :]
        return (a_acc, b_acc)

    return lax.fori_loop(base + EW, chi + 1, it, (a_pe, b_pe))


def _gate_body(tg_ref, ids_ref, ah_ref, al_ref, bh_ref, bl_ref,
               a_ref, b_ref, wm_ref, ns_ref):
    ids = ids_ref[...]
    a_pe, b_pe = _expand_tables(ids, ah_ref, al_ref, bh_ref, bl_ref,
                                a_ref, b_ref, D1)
    tgn = tg_ref[...] * a_pe + b_pe
    filt = jnp.tanh(jnp.dot(tgn, wm_ref[...]))
    ns_ref[...] = (jax.nn.relu(tgn) * filt)[:, :NF]


def _tc_gate(tg, ids_col, tabs1, wm_pad):
    nb = NE // TBX
    eb = lambda i: (i, 0)
    wb = lambda i: (0, 0)
    return pl.pallas_call(
        _gate_body,
        grid=(nb,),
        in_specs=[
            pl.BlockSpec((TBX, D1), eb),
            pl.BlockSpec((TBX, 1), eb),
            pl.BlockSpec((NC, D1), wb),
            pl.BlockSpec((NC, D1), wb),
            pl.BlockSpec((NC, D1), wb),
            pl.BlockSpec((NC, D1), wb),
            pl.BlockSpec((NC, D1), wb),
            pl.BlockSpec((NC, D1), wb),
            pl.BlockSpec((D1, 1), wb),
        ],
        out_specs=pl.BlockSpec((TBX, NF), eb),
        out_shape=jax.ShapeDtypeStruct((NE, NF), F32),
    )(tg, ids_col, tabs1[2], tabs1[3], tabs1[4], tabs1[5],
      tabs1[0], tabs1[1], wm_pad)


def _final_body(ns_ref, ids_ref, ah_ref, al_ref, bh_ref, bl_ref,
                a_ref, b_ref, e_ref,
                w1a_ref, b1a_ref, w2a_ref, b2a_ref,
                w1b_ref, b1b_ref, w2b_ref, b2b_ref, o_ref):
    ids = ids_ref[...]
    a_pe, b_pe = _expand_tables(ids, ah_ref, al_ref, bh_ref, bl_ref,
                                a_ref, b_ref, NF)
    x = ns_ref[...] * a_pe + b_pe
    h = jnp.dot(jax.nn.relu(jnp.dot(x, w1a_ref[...]) + b1a_ref[...]),
                w2a_ref[...]) + b2a_ref[...]
    x = x + h
    h = jnp.dot(jax.nn.relu(jnp.dot(x, w1b_ref[...]) + b1b_ref[...]),
                w2b_ref[...]) + b2b_ref[...]
    x = x + h
    o_ref[...] = INV_SQRT_2 * jax.nn.relu(e_ref[...] + x)


def _tc_final(ns, ids_col, tabs2, edge, rw):
    nb = NE // TBX
    mid = NF // 2
    eb = lambda i: (i, 0)
    wb = lambda i: (0, 0)
    return pl.pallas_call(
        _final_body,
        grid=(nb,),
        in_specs=[
            pl.BlockSpec((TBX, NF), eb),
            pl.BlockSpec((TBX, 1), eb),
            pl.BlockSpec((NC, NF), wb),
            pl.BlockSpec((NC, NF), wb),
            pl.BlockSpec((NC, NF), wb),
            pl.BlockSpec((NC, NF), wb),
            pl.BlockSpec((NC, NF), wb),
            pl.BlockSpec((NC, NF), wb),
            pl.BlockSpec((TBX, NF), eb),
            pl.BlockSpec((NF, mid), wb),
            pl.BlockSpec((1, mid), wb),
            pl.BlockSpec((mid, NF), wb),
            pl.BlockSpec((1, NF), wb),
            pl.BlockSpec((NF, mid), wb),
            pl.BlockSpec((1, mid), wb),
            pl.BlockSpec((mid, NF), wb),
            pl.BlockSpec((1, NF), wb),
        ],
        out_specs=pl.BlockSpec((TBX, NF), eb),
        out_shape=jax.ShapeDtypeStruct((NE, NF), F32),
    )(ns, ids_col, tabs2[2], tabs2[3], tabs2[4], tabs2[5],
      tabs2[0], tabs2[1], edge, *rw)


def kernel(atom_fea, edge, crystal_atom_idx, crystal_edge_idx, nbr_fea_idx,
           rbf, W_full, W_mask, res_W1a, res_b1a, res_W2a, res_b2a,
           res_W1b, res_b1b, res_W2b, res_b2b, gamma1, beta1, gamma2, beta2):
    ids_i32 = crystal_edge_idx.astype(jnp.int32)
    ids_2d = ids_i32.reshape(NE // GB, GB)
    ids_col = ids_i32.reshape(NE, 1)
    nbr_t = nbr_fea_idx.astype(jnp.int32).T
    idx0 = nbr_t[0].reshape(NE // GB, GB)
    idx1 = nbr_t[1].reshape(NE // GB, GB)
    w1 = W_full[:AF, :]
    w2 = W_full[AF:, :]
    wm_pad = jnp.concatenate([jnp.zeros((NF, 1), F32), W_mask], axis=0)

    proj = _tc_proj(atom_fea, w1)
    ep = _tc_ep(edge, w2)
    tg, s1, q1, cnt = _make_gather_tg_stats()(proj, idx0, idx1, ids_2d, ep)

    tabs1 = _tc_finalize(D1, s1, q1, cnt, gamma1, beta1)
    ns = _tc_gate(tg, ids_col, tabs1, wm_pad)

    s2, q2 = _make_stats(NF)(ns, ids_2d)
    tabs2 = _tc_finalize(NF, s2, q2, cnt, gamma2, beta2)

    rw = (res_W1a, res_b1a.reshape(1, -1), res_W2a, res_b2a.reshape(1, -1),
          res_W1b, res_b1b.reshape(1, -1), res_W2b, res_b2b.reshape(1, -1))
    return _tc_final(ns, ids_col, tabs2, edge, rw)


# speedup vs baseline: 2.1680x; 1.1296x over previous
import functools

import jax
import jax.numpy as jnp
from jax import lax
from jax.experimental import pallas as pl
from jax.experimental.pallas import tpu as pltpu
from jax.experimental.pallas import tpu_sc as plsc

F32 = jnp.float32
AF = 128
NF = 16
D1 = 2 * NF
NN = 10000
NE = 320000
NC = 1000
EPS = 1e-5
INV_SQRT_2 = 1.0 / 2.0 ** 0.5

GB = 128
TB = 6400
TBX = 2560
EW = 128

_HIGH = jax.lax.Precision.HIGHEST


def _mesh():
    return plsc.VectorSubcoreMesh(core_axis_name="c", subcore_axis_name="s")


_SC_PARAMS = pltpu.CompilerParams(use_tc_tiling_on_sc=False)


def _proj_body(x_ref, w_ref, o_ref):
    o_ref[...] = jnp.dot(x_ref[...], w_ref[...], precision=_HIGH)


def _tc_proj(atom_fea, w1):
    return pl.pallas_call(
        _proj_body,
        out_shape=jax.ShapeDtypeStruct((NN, D1), F32),
    )(atom_fea, w1)


def _ep_body(e_ref, w_ref, o_ref):
    o_ref[...] = jnp.dot(e_ref[...], w_ref[...])


def _tc_ep(edge, w2):
    nb = NE // TB
    return pl.pallas_call(
        _ep_body,
        grid=(nb,),
        in_specs=[
            pl.BlockSpec((TB, NF), lambda i: (i, 0)),
            pl.BlockSpec((NF, D1), lambda i: (0, 0)),
        ],
        out_specs=pl.BlockSpec((TB, D1), lambda i: (i, 0)),
        out_shape=jax.ShapeDtypeStruct((NE, D1), F32),
    )(edge, w2)


SB = 512
SUB = SB // GB


def _make_gather_tg_stats():
    @functools.partial(
        pl.kernel,
        out_type=(
            jax.ShapeDtypeStruct((NE, D1), F32),
            jax.ShapeDtypeStruct((2, NC, D1), F32),
            jax.ShapeDtypeStruct((2, NC, D1), F32),
            jax.ShapeDtypeStruct((2, NC, 16), F32),
        ),
        mesh=_mesh(),
        compiler_params=_SC_PARAMS,
        scratch_types=[
            pltpu.VMEM((SB, D1), F32),
            pltpu.VMEM((SB, D1), F32),
            pltpu.VMEM((SB, D1), F32),
            pltpu.VMEM((GB, 16), F32),
            pltpu.VMEM((125, D1), F32),
            pltpu.VMEM_SHARED((NC, D1), F32),
            pltpu.VMEM_SHARED((NC, D1), F32),
            pltpu.VMEM_SHARED((NC, 16), F32),
            pltpu.SemaphoreType.DMA,
            pltpu.SemaphoreType.DMA,
        ],
    )
    def gather_tg(proj_hbm, idx0_hbm, idx1_hbm, ids_hbm, ep_hbm,
                  tg_hbm, sum_hbm, sq_hbm, cnt_hbm,
                  p0_v, p1_v, sq_v, ones_v, z_v, ssum, ssq, scnt,
                  gsem, wsem):
        cid = lax.axis_index("c")
        sid = lax.axis_index("s")

        @pl.when(sid == 0)
        def _():
            @pl.loop(0, 125)
            def _(r):
                @pl.loop(0, D1, step=16)
                def _(c):
                    z_v.at[pl.ds(r, 1), pl.ds(c, 16)][...] = jnp.zeros(
                        (1, 16), F32)

            @pl.loop(0, 8)
            def _(k):
                pltpu.sync_copy(z_v, ssum.at[pl.ds(k * 125, 125)])
                pltpu.sync_copy(z_v, ssq.at[pl.ds(k * 125, 125)])
                pltpu.sync_copy(z_v.at[:, pl.ds(0, 16)],
                                scnt.at[pl.ds(k * 125, 125)])

        @pl.loop(0, GB)
        def _(r):
            ones_v.at[pl.ds(r, 1), pl.ds(0, 16)][...] = jnp.ones((1, 16), F32)

        plsc.subcore_barrier()

        def body(idx0_v, idx1_v, ids_v, ep_v, tg_v):
            cps = []
            for k in range(SUB):
                dst = pl.ds(k * GB, GB)
                cps.append(pltpu.async_copy(
                    proj_hbm.at[idx0_v.at[k]], p0_v.at[dst], gsem))
                cps.append(pltpu.async_copy(
                    proj_hbm.at[idx1_v.at[k]], p1_v.at[dst], gsem))
            for cp in cps:
                cp.wait()

            @pl.loop(0, SB)
            def _(r):
                @pl.loop(0, D1, step=16)
                def _(c):
                    slc = (pl.ds(r, 1), pl.ds(c, 16))
                    v = (p1_v.at[slc][...] - p0_v.at[slc][...]
                         + ep_v.at[slc][...])
                    tg_v.at[slc][...] = v
                    sq_v.at[slc][...] = v * v

            cps = []
            for k in range(SUB):
                src = pl.ds(k * GB, GB)
                cps.append(pltpu.async_copy(
                    tg_v.at[src], ssum.at[ids_v.at[k]], wsem, add=True))
                cps.append(pltpu.async_copy(
                    sq_v.at[src], ssq.at[ids_v.at[k]], wsem, add=True))
                cps.append(pltpu.async_copy(
                    ones_v, scnt.at[ids_v.at[k]], wsem, add=True))
            for cp in cps:
                cp.wait()

        pltpu.emit_pipeline(
            body,
            grid=(NE // SB,),
            in_specs=[
                pl.BlockSpec((SUB, GB), lambda i: (i, 0)),
                pl.BlockSpec((SUB, GB), lambda i: (i, 0)),
                pl.BlockSpec((SUB, GB), lambda i: (i, 0)),
                pl.BlockSpec((SB, D1), lambda i: (i, 0)),
            ],
            out_specs=[pl.BlockSpec((SB, D1), lambda i: (i, 0))],
            core_axis_name=("c", "s"),
            dimension_semantics=(pltpu.PARALLEL,),
        )(idx0_hbm, idx1_hbm, ids_hbm, ep_hbm, tg_hbm)

        plsc.subcore_barrier()

        @pl.when(sid == 0)
        def _():
            pltpu.sync_copy(ssum, sum_hbm.at[cid])
            pltpu.sync_copy(ssq, sq_hbm.at[cid])
            pltpu.sync_copy(scnt, cnt_hbm.at[cid])

    return gather_tg


def _make_stats(d):
    @functools.partial(
        pl.kernel,
        out_type=(
            jax.ShapeDtypeStruct((2, NC, d), F32),
            jax.ShapeDtypeStruct((2, NC, d), F32),
        ),
        mesh=_mesh(),
        compiler_params=_SC_PARAMS,
        scratch_types=[
            pltpu.VMEM((SB, d), F32),
            pltpu.VMEM((125, d), F32),
            pltpu.VMEM_SHARED((NC, d), F32),
            pltpu.VMEM_SHARED((NC, d), F32),
            pltpu.SemaphoreType.DMA,
        ],
    )
    def stats(x_hbm, ids_hbm, sum_hbm, sq_hbm, sq_v, z_v, ssum, ssq, wsem):
        cid = lax.axis_index("c")
        sid = lax.axis_index("s")

        @pl.when(sid == 0)
        def _():
            @pl.loop(0, 125)
            def _(r):
                @pl.loop(0, d, step=16)
                def _(c):
                    z_v.at[pl.ds(r, 1), pl.ds(c, 16)][...] = jnp.zeros(
                        (1, 16), F32)

            @pl.loop(0, 8)
            def _(k):
                pltpu.sync_copy(z_v, ssum.at[pl.ds(k * 125, 125)])
                pltpu.sync_copy(z_v, ssq.at[pl.ds(k * 125, 125)])

        plsc.subcore_barrier()

        def body(x_v, ids_v):
            @pl.loop(0, SB)
            def _(r):
                @pl.loop(0, d, step=16)
                def _(c):
                    slc = (pl.ds(r, 1), pl.ds(c, 16))
                    v = x_v.at[slc][...]
                    sq_v.at[slc][...] = v * v

            cps = []
            for k in range(SUB):
                src = pl.ds(k * GB, GB)
                cps.append(pltpu.async_copy(
                    x_v.at[src], ssum.at[ids_v.at[k]], wsem, add=True))
                cps.append(pltpu.async_copy(
                    sq_v.at[src], ssq.at[ids_v.at[k]], wsem, add=True))
            for cp in cps:
                cp.wait()

        pltpu.emit_pipeline(
            body,
            grid=(NE // SB,),
            in_specs=[
                pl.BlockSpec((SB, d), lambda i: (i, 0)),
                pl.BlockSpec((SUB, GB), lambda i: (i, 0)),
            ],
            out_specs=[],
            core_axis_name=("c", "s"),
            dimension_semantics=(pltpu.PARALLEL,),
        )(x_hbm, ids_hbm)

        plsc.subcore_barrier()

        @pl.when(sid == 0)
        def _():
            pltpu.sync_copy(ssum, sum_hbm.at[cid])
            pltpu.sync_copy(ssq, sq_hbm.at[cid])

    return stats


def _fin_body(sum_ref, sq_ref, cnt_ref, g_ref, bt_ref,
              a_ref, b_ref, ah_ref, al_ref, bh_ref, bl_ref):
    s = sum_ref[0] + sum_ref[1]
    q = sq_ref[0] + sq_ref[1]
    n = jnp.maximum(cnt_ref[0, :, 0:1] + cnt_ref[1, :, 0:1], 1.0)
    mean = s / n
    var = jnp.maximum(q / n - mean * mean, 0.0)
    a = g_ref[...] * lax.rsqrt(var + EPS)
    b = bt_ref[...] - mean * a
    a_ref[...] = a
    b_ref[...] = b
    ah = a.astype(jnp.bfloat16)
    bh = b.astype(jnp.bfloat16)
    ah_ref[...] = ah
    bh_ref[...] = bh
    al_ref[...] = (a - ah.astype(F32)).astype(jnp.bfloat16)
    bl_ref[...] = (b - bh.astype(F32)).astype(jnp.bfloat16)


def _tc_finalize(d, ssum, ssq, cnt, gamma, beta):
    bf = jnp.bfloat16
    return pl.pallas_call(
        _fin_body,
        out_shape=(
            jax.ShapeDtypeStruct((NC, d), F32),
            jax.ShapeDtypeStruct((NC, d), F32),
            jax.ShapeDtypeStruct((NC, d), bf),
            jax.ShapeDtypeStruct((NC, d), bf),
            jax.ShapeDtypeStruct((NC, d), bf),
            jax.ShapeDtypeStruct((NC, d), bf),
        ),
    )(ssum, ssq, cnt, gamma.reshape(1, d), beta.reshape(1, d))


def _expand_tables(ids, ah_ref, al_ref, bh_ref, bl_ref, a_ref, b_ref, d):
    n = ids.shape[0]
    clo = jnp.min(ids)
    chi = jnp.max(ids)
    base = pl.multiple_of(jnp.minimum((clo // 8) * 8, NC - EW), 8)
    lid = ids - base
    iota = lax.broadcasted_iota(jnp.int32, (1, EW), 1)
    oh = (lid == iota).astype(jnp.bfloat16)
    wah = ah_ref[pl.ds(base, EW), :]
    wal = al_ref[pl.ds(base, EW), :]
    wbh = bh_ref[pl.ds(base, EW), :]
    wbl = bl_ref[pl.ds(base, EW), :]
    a_pe = (jnp.dot(oh, wah, preferred_element_type=F32)
            + jnp.dot(oh, wal, preferred_element_type=F32))
    b_pe = (jnp.dot(oh, wbh, preferred_element_type=F32)
            + jnp.dot(oh, wbl, preferred_element_type=F32))

    def it(c, ab):
        a_acc, b_acc = ab
        m = (ids == c).astype(F32)
        a_acc = a_acc + m * a_ref[pl.ds(c, 1), :]
        b_acc = b_acc + m * b_ref[pl.ds(c, 1), :]
        return (a_acc, b_acc)

    return lax.fori_loop(base + EW, chi + 1, it, (a_pe, b_pe))


def _gate_body(tg_ref, ids_ref, ah_ref, al_ref, bh_ref, bl_ref,
               a_ref, b_ref, wm_ref, ns_ref):
    ids = ids_ref[...]
    a_pe, b_pe = _expand_tables(ids, ah_ref, al_ref, bh_ref, bl_ref,
                                a_ref, b_ref, D1)
    tgn = tg_ref[...] * a_pe + b_pe
    filt = jnp.tanh(jnp.dot(tgn, wm_ref[...]))
    ns_ref[...] = (jax.nn.relu(tgn) * filt)[:, :NF]


def _tc_gate(tg, ids_col, tabs1, wm_pad):
    nb = NE // TBX
    eb = lambda i: (i, 0)
    wb = lambda i: (0, 0)
    return pl.pallas_call(
        _gate_body,
        grid=(nb,),
        in_specs=[
            pl.BlockSpec((TBX, D1), eb),
            pl.BlockSpec((TBX, 1), eb),
            pl.BlockSpec((NC, D1), wb),
            pl.BlockSpec((NC, D1), wb),
            pl.BlockSpec((NC, D1), wb),
            pl.BlockSpec((NC, D1), wb),
            pl.BlockSpec((NC, D1), wb),
            pl.BlockSpec((NC, D1), wb),
            pl.BlockSpec((D1, 1), wb),
        ],
        out_specs=pl.BlockSpec((TBX, NF), eb),
        out_shape=jax.ShapeDtypeStruct((NE, NF), F32),
    )(tg, ids_col, tabs1[2], tabs1[3], tabs1[4], tabs1[5],
      tabs1[0], tabs1[1], wm_pad)


def _final_body(ns_ref, ids_ref, ah_ref, al_ref, bh_ref, bl_ref,
                a_ref, b_ref, e_ref,
                w1a_ref, b1a_ref, w2a_ref, b2a_ref,
                w1b_ref, b1b_ref, w2b_ref, b2b_ref, o_ref):
    ids = ids_ref[...]
    a_pe, b_pe = _expand_tables(ids, ah_ref, al_ref, bh_ref, bl_ref,
                                a_ref, b_ref, NF)
    x = ns_ref[...] * a_pe + b_pe
    h = jnp.dot(jax.nn.relu(jnp.dot(x, w1a_ref[...]) + b1a_ref[...]),
                w2a_ref[...]) + b2a_ref[...]
    x = x + h
    h = jnp.dot(jax.nn.relu(jnp.dot(x, w1b_ref[...]) + b1b_ref[...]),
                w2b_ref[...]) + b2b_ref[...]
    x = x + h
    o_ref[...] = INV_SQRT_2 * jax.nn.relu(e_ref[...] + x)


def _tc_final(ns, ids_col, tabs2, edge, rw):
    nb = NE // TBX
    mid = NF // 2
    eb = lambda i: (i, 0)
    wb = lambda i: (0, 0)
    return pl.pallas_call(
        _final_body,
        grid=(nb,),
        in_specs=[
            pl.BlockSpec((TBX, NF), eb),
            pl.BlockSpec((TBX, 1), eb),
            pl.BlockSpec((NC, NF), wb),
            pl.BlockSpec((NC, NF), wb),
            pl.BlockSpec((NC, NF), wb),
            pl.BlockSpec((NC, NF), wb),
            pl.BlockSpec((NC, NF), wb),
            pl.BlockSpec((NC, NF), wb),
            pl.BlockSpec((TBX, NF), eb),
            pl.BlockSpec((NF, mid), wb),
            pl.BlockSpec((1, mid), wb),
            pl.BlockSpec((mid, NF), wb),
            pl.BlockSpec((1, NF), wb),
            pl.BlockSpec((NF, mid), wb),
            pl.BlockSpec((1, mid), wb),
            pl.BlockSpec((mid, NF), wb),
            pl.BlockSpec((1, NF), wb),
        ],
        out_specs=pl.BlockSpec((TBX, NF), eb),
        out_shape=jax.ShapeDtypeStruct((NE, NF), F32),
    )(ns, ids_col, tabs2[2], tabs2[3], tabs2[4], tabs2[5],
      tabs2[0], tabs2[1], edge, *rw)


def kernel(atom_fea, edge, crystal_atom_idx, crystal_edge_idx, nbr_fea_idx,
           rbf, W_full, W_mask, res_W1a, res_b1a, res_W2a, res_b2a,
           res_W1b, res_b1b, res_W2b, res_b2b, gamma1, beta1, gamma2, beta2):
    ids_i32 = crystal_edge_idx.astype(jnp.int32)
    ids_2d = ids_i32.reshape(NE // GB, GB)
    ids_col = ids_i32.reshape(NE, 1)
    nbr_t = nbr_fea_idx.astype(jnp.int32).T
    idx0 = nbr_t[0].reshape(NE // GB, GB)
    idx1 = nbr_t[1].reshape(NE // GB, GB)
    w1 = W_full[:AF, :]
    w2 = W_full[AF:, :]
    wm_pad = jnp.concatenate([jnp.zeros((NF, 1), F32), W_mask], axis=0)

    proj = _tc_proj(atom_fea, w1)
    ep = _tc_ep(edge, w2)
    tg, s1, q1, cnt = _make_gather_tg_stats()(proj, idx0, idx1, ids_2d, ep)

    tabs1 = _tc_finalize(D1, s1, q1, cnt, gamma1, beta1)
    ns = _tc_gate(tg, ids_col, tabs1, wm_pad)

    s2, q2 = _make_stats(NF)(ns, ids_2d)
    tabs2 = _tc_finalize(NF, s2, q2, cnt, gamma2, beta2)

    rw = (res_W1a, res_b1a.reshape(1, -1), res_W2a, res_b2a.reshape(1, -1),
          res_W1b, res_b1b.reshape(1, -1), res_W2b, res_b2b.reshape(1, -1))
    return _tc_final(ns, ids_col, tabs2, edge, rw)


# speedup vs baseline: 2.3024x; 1.0620x over previous
import functools

import jax
import jax.numpy as jnp
from jax import lax
from jax.experimental import pallas as pl
from jax.experimental.pallas import tpu as pltpu
from jax.experimental.pallas import tpu_sc as plsc

F32 = jnp.float32
AF = 128
NF = 16
D1 = 2 * NF
NN = 10000
NE = 320000
NC = 1000
EPS = 1e-5
INV_SQRT_2 = 1.0 / 2.0 ** 0.5

GB = 128
TB = 6400
TBX = 6400
EW = 128

_HIGH = jax.lax.Precision.HIGHEST


def _mesh():
    return plsc.VectorSubcoreMesh(core_axis_name="c", subcore_axis_name="s")


_SC_PARAMS = pltpu.CompilerParams(use_tc_tiling_on_sc=False)


def _proj_body(x_ref, w_ref, o_ref):
    o_ref[...] = jnp.dot(x_ref[...], w_ref[...], precision=_HIGH)


def _tc_proj(atom_fea, w1):
    return pl.pallas_call(
        _proj_body,
        out_shape=jax.ShapeDtypeStruct((NN, D1), F32),
    )(atom_fea, w1)


def _ep_body(e_ref, w_ref, o_ref):
    o_ref[...] = jnp.dot(e_ref[...], w_ref[...])


def _tc_ep(edge, w2):
    nb = NE // TB
    return pl.pallas_call(
        _ep_body,
        grid=(nb,),
        in_specs=[
            pl.BlockSpec((TB, NF), lambda i: (i, 0)),
            pl.BlockSpec((NF, D1), lambda i: (0, 0)),
        ],
        out_specs=pl.BlockSpec((TB, D1), lambda i: (i, 0)),
        out_shape=jax.ShapeDtypeStruct((NE, D1), F32),
    )(edge, w2)


SB = 512
SUB = SB // GB


def _make_gather_tg_stats():
    @functools.partial(
        pl.kernel,
        out_type=(
            jax.ShapeDtypeStruct((NE, D1), F32),
            jax.ShapeDtypeStruct((2, NC, D1), F32),
            jax.ShapeDtypeStruct((2, NC, D1), F32),
            jax.ShapeDtypeStruct((2, NC, 16), F32),
        ),
        mesh=_mesh(),
        compiler_params=_SC_PARAMS,
        scratch_types=[
            pltpu.VMEM((SB, D1), F32),
            pltpu.VMEM((SB, D1), F32),
            pltpu.VMEM((SB, D1), F32),
            pltpu.VMEM((GB, 16), F32),
            pltpu.VMEM((125, D1), F32),
            pltpu.VMEM_SHARED((NC, D1), F32),
            pltpu.VMEM_SHARED((NC, D1), F32),
            pltpu.VMEM_SHARED((NC, 16), F32),
            pltpu.SemaphoreType.DMA,
            pltpu.SemaphoreType.DMA,
        ],
    )
    def gather_tg(proj_hbm, idx0_hbm, idx1_hbm, ids_hbm, ep_hbm,
                  tg_hbm, sum_hbm, sq_hbm, cnt_hbm,
                  p0_v, p1_v, sq_v, ones_v, z_v, ssum, ssq, scnt,
                  gsem, wsem):
        cid = lax.axis_index("c")
        sid = lax.axis_index("s")

        @pl.when(sid == 0)
        def _():
            @pl.loop(0, 125)
            def _(r):
                @pl.loop(0, D1, step=16)
                def _(c):
                    z_v.at[pl.ds(r, 1), pl.ds(c, 16)][...] = jnp.zeros(
                        (1, 16), F32)

            @pl.loop(0, 8)
            def _(k):
                pltpu.sync_copy(z_v, ssum.at[pl.ds(k * 125, 125)])
                pltpu.sync_copy(z_v, ssq.at[pl.ds(k * 125, 125)])
                pltpu.sync_copy(z_v.at[:, pl.ds(0, 16)],
                                scnt.at[pl.ds(k * 125, 125)])

        @pl.loop(0, GB)
        def _(r):
            ones_v.at[pl.ds(r, 1), pl.ds(0, 16)][...] = jnp.ones((1, 16), F32)

        plsc.subcore_barrier()

        def body(idx0_v, idx1_v, ids_v, ep_v, tg_v):
            cps = []
            for k in range(SUB):
                dst = pl.ds(k * GB, GB)
                cps.append(pltpu.async_copy(
                    proj_hbm.at[idx0_v.at[k]], p0_v.at[dst], gsem))
                cps.append(pltpu.async_copy(
                    proj_hbm.at[idx1_v.at[k]], p1_v.at[dst], gsem))
            for cp in cps:
                cp.wait()

            @pl.loop(0, SB)
            def _(r):
                @pl.loop(0, D1, step=16)
                def _(c):
                    slc = (pl.ds(r, 1), pl.ds(c, 16))
                    v = (p1_v.at[slc][...] - p0_v.at[slc][...]
                         + ep_v.at[slc][...])
                    tg_v.at[slc][...] = v
                    sq_v.at[slc][...] = v * v

            cps = []
            for k in range(SUB):
                src = pl.ds(k * GB, GB)
                cps.append(pltpu.async_copy(
                    tg_v.at[src], ssum.at[ids_v.at[k]], wsem, add=True))
                cps.append(pltpu.async_copy(
                    sq_v.at[src], ssq.at[ids_v.at[k]], wsem, add=True))
                cps.append(pltpu.async_copy(
                    ones_v, scnt.at[ids_v.at[k]], wsem, add=True))
            for cp in cps:
                cp.wait()

        pltpu.emit_pipeline(
            body,
            grid=(NE // SB,),
            in_specs=[
                pl.BlockSpec((SUB, GB), lambda i: (i, 0)),
                pl.BlockSpec((SUB, GB), lambda i: (i, 0)),
                pl.BlockSpec((SUB, GB), lambda i: (i, 0)),
                pl.BlockSpec((SB, D1), lambda i: (i, 0)),
            ],
            out_specs=[pl.BlockSpec((SB, D1), lambda i: (i, 0))],
            core_axis_name=("c", "s"),
            dimension_semantics=(pltpu.PARALLEL,),
        )(idx0_hbm, idx1_hbm, ids_hbm, ep_hbm, tg_hbm)

        plsc.subcore_barrier()

        @pl.when(sid == 0)
        def _():
            pltpu.sync_copy(ssum, sum_hbm.at[cid])
            pltpu.sync_copy(ssq, sq_hbm.at[cid])
            pltpu.sync_copy(scnt, cnt_hbm.at[cid])

    return gather_tg


def _make_stats(d):
    @functools.partial(
        pl.kernel,
        out_type=(
            jax.ShapeDtypeStruct((2, NC, d), F32),
            jax.ShapeDtypeStruct((2, NC, d), F32),
        ),
        mesh=_mesh(),
        compiler_params=_SC_PARAMS,
        scratch_types=[
            pltpu.VMEM((SB, d), F32),
            pltpu.VMEM((125, d), F32),
            pltpu.VMEM_SHARED((NC, d), F32),
            pltpu.VMEM_SHARED((NC, d), F32),
            pltpu.SemaphoreType.DMA,
        ],
    )
    def stats(x_hbm, ids_hbm, sum_hbm, sq_hbm, sq_v, z_v, ssum, ssq, wsem):
        cid = lax.axis_index("c")
        sid = lax.axis_index("s")

        @pl.when(sid == 0)
        def _():
            @pl.loop(0, 125)
            def _(r):
                @pl.loop(0, d, step=16)
                def _(c):
                    z_v.at[pl.ds(r, 1), pl.ds(c, 16)][...] = jnp.zeros(
                        (1, 16), F32)

            @pl.loop(0, 8)
            def _(k):
                pltpu.sync_copy(z_v, ssum.at[pl.ds(k * 125, 125)])
                pltpu.sync_copy(z_v, ssq.at[pl.ds(k * 125, 125)])

        plsc.subcore_barrier()

        def body(x_v, ids_v):
            @pl.loop(0, SB)
            def _(r):
                @pl.loop(0, d, step=16)
                def _(c):
                    slc = (pl.ds(r, 1), pl.ds(c, 16))
                    v = x_v.at[slc][...]
                    sq_v.at[slc][...] = v * v

            cps = []
            for k in range(SUB):
                src = pl.ds(k * GB, GB)
                cps.append(pltpu.async_copy(
                    x_v.at[src], ssum.at[ids_v.at[k]], wsem, add=True))
                cps.append(pltpu.async_copy(
                    sq_v.at[src], ssq.at[ids_v.at[k]], wsem, add=True))
            for cp in cps:
                cp.wait()

        pltpu.emit_pipeline(
            body,
            grid=(NE // SB,),
            in_specs=[
                pl.BlockSpec((SB, d), lambda i: (i, 0)),
                pl.BlockSpec((SUB, GB), lambda i: (i, 0)),
            ],
            out_specs=[],
            core_axis_name=("c", "s"),
            dimension_semantics=(pltpu.PARALLEL,),
        )(x_hbm, ids_hbm)

        plsc.subcore_barrier()

        @pl.when(sid == 0)
        def _():
            pltpu.sync_copy(ssum, sum_hbm.at[cid])
            pltpu.sync_copy(ssq, sq_hbm.at[cid])

    return stats


def _fin_body(sum_ref, sq_ref, cnt_ref, g_ref, bt_ref,
              a_ref, b_ref, ah_ref, al_ref, bh_ref, bl_ref):
    s = sum_ref[0] + sum_ref[1]
    q = sq_ref[0] + sq_ref[1]
    n = jnp.maximum(cnt_ref[0, :, 0:1] + cnt_ref[1, :, 0:1], 1.0)
    mean = s / n
    var = jnp.maximum(q / n - mean * mean, 0.0)
    a = g_ref[...] * lax.rsqrt(var + EPS)
    b = bt_ref[...] - mean * a
    a_ref[...] = a
    b_ref[...] = b
    ah = a.astype(jnp.bfloat16)
    bh = b.astype(jnp.bfloat16)
    ah_ref[...] = ah
    bh_ref[...] = bh
    al_ref[...] = (a - ah.astype(F32)).astype(jnp.bfloat16)
    bl_ref[...] = (b - bh.astype(F32)).astype(jnp.bfloat16)


def _tc_finalize(d, ssum, ssq, cnt, gamma, beta):
    bf = jnp.bfloat16
    return pl.pallas_call(
        _fin_body,
        out_shape=(
            jax.ShapeDtypeStruct((NC, d), F32),
            jax.ShapeDtypeStruct((NC, d), F32),
            jax.ShapeDtypeStruct((NC, d), bf),
            jax.ShapeDtypeStruct((NC, d), bf),
            jax.ShapeDtypeStruct((NC, d), bf),
            jax.ShapeDtypeStruct((NC, d), bf),
        ),
    )(ssum, ssq, cnt, gamma.reshape(1, d), beta.reshape(1, d))


def _expand_tables(ids, ah_ref, al_ref, bh_ref, bl_ref, a_ref, b_ref, d):
    n = ids.shape[0]
    clo = jnp.min(ids)
    chi = jnp.max(ids)
    base = pl.multiple_of(jnp.minimum((clo // 8) * 8, NC - EW), 8)
    lid = ids - base
    iota = lax.broadcasted_iota(jnp.int32, (1, EW), 1)
    oh = (lid == iota).astype(jnp.bfloat16)
    wah = ah_ref[pl.ds(base, EW), :]
    wal = al_ref[pl.ds(base, EW), :]
    wbh = bh_ref[pl.ds(base, EW), :]
    wbl = bl_ref[pl.ds(base, EW), :]
    a_pe = (jnp.dot(oh, wah, preferred_element_type=F32)
            + jnp.dot(oh, wal, preferred_element_type=F32))
    b_pe = (jnp.dot(oh, wbh, preferred_element_type=F32)
            + jnp.dot(oh, wbl, preferred_element_type=F32))

    def it(c, ab):
        a_acc, b_acc = ab
        m = (ids == c).astype(F32)
        a_acc = a_acc + m * a_ref[pl.ds(c, 1), :]
        b_acc = b_acc + m * b_ref[pl.ds(c, 1), :]
        return (a_acc, b_acc)

    return lax.fori_loop(base + EW, chi + 1, it, (a_pe, b_pe))


def _gate_body(tg_ref, ids_ref, ah_ref, al_ref, bh_ref, bl_ref,
               a_ref, b_ref, wm_ref, ns_ref):
    ids = ids_ref[...]
    a_pe, b_pe = _expand_tables(ids, ah_ref, al_ref, bh_ref, bl_ref,
                                a_ref, b_ref, D1)
    tgn = tg_ref[...] * a_pe + b_pe
    filt = jnp.tanh(jnp.dot(tgn, wm_ref[...]))
    ns_ref[...] = (jax.nn.relu(tgn) * filt)[:, :NF]


def _tc_gate(tg, ids_col, tabs1, wm_pad):
    nb = NE // TBX
    eb = lambda i: (i, 0)
    wb = lambda i: (0, 0)
    return pl.pallas_call(
        _gate_body,
        grid=(nb,),
        in_specs=[
            pl.BlockSpec((TBX, D1), eb),
            pl.BlockSpec((TBX, 1), eb),
            pl.BlockSpec((NC, D1), wb),
            pl.BlockSpec((NC, D1), wb),
            pl.BlockSpec((NC, D1), wb),
            pl.BlockSpec((NC, D1), wb),
            pl.BlockSpec((NC, D1), wb),
            pl.BlockSpec((NC, D1), wb),
            pl.BlockSpec((D1, 1), wb),
        ],
        out_specs=pl.BlockSpec((TBX, NF), eb),
        out_shape=jax.ShapeDtypeStruct((NE, NF), F32),
    )(tg, ids_col, tabs1[2], tabs1[3], tabs1[4], tabs1[5],
      tabs1[0], tabs1[1], wm_pad)


def _final_body(ns_ref, ids_ref, ah_ref, al_ref, bh_ref, bl_ref,
                a_ref, b_ref, e_ref,
                w1a_ref, b1a_ref, w2a_ref, b2a_ref,
                w1b_ref, b1b_ref, w2b_ref, b2b_ref, o_ref):
    ids = ids_ref[...]
    a_pe, b_pe = _expand_tables(ids, ah_ref, al_ref, bh_ref, bl_ref,
                                a_ref, b_ref, NF)
    x = ns_ref[...] * a_pe + b_pe
    h = jnp.dot(jax.nn.relu(jnp.dot(x, w1a_ref[...]) + b1a_ref[...]),
                w2a_ref[...]) + b2a_ref[...]
    x = x + h
    h = jnp.dot(jax.nn.relu(jnp.dot(x, w1b_ref[...]) + b1b_ref[...]),
                w2b_ref[...]) + b2b_ref[...]
    x = x + h
    o_ref[...] = INV_SQRT_2 * jax.nn.relu(e_ref[...] + x)


def _tc_final(ns, ids_col, tabs2, edge, rw):
    nb = NE // TBX
    mid = NF // 2
    eb = lambda i: (i, 0)
    wb = lambda i: (0, 0)
    return pl.pallas_call(
        _final_body,
        grid=(nb,),
        in_specs=[
            pl.BlockSpec((TBX, NF), eb),
            pl.BlockSpec((TBX, 1), eb),
            pl.BlockSpec((NC, NF), wb),
            pl.BlockSpec((NC, NF), wb),
            pl.BlockSpec((NC, NF), wb),
            pl.BlockSpec((NC, NF), wb),
            pl.BlockSpec((NC, NF), wb),
            pl.BlockSpec((NC, NF), wb),
            pl.BlockSpec((TBX, NF), eb),
            pl.BlockSpec((NF, mid), wb),
            pl.BlockSpec((1, mid), wb),
            pl.BlockSpec((mid, NF), wb),
            pl.BlockSpec((1, NF), wb),
            pl.BlockSpec((NF, mid), wb),
            pl.BlockSpec((1, mid), wb),
            pl.BlockSpec((mid, NF), wb),
            pl.BlockSpec((1, NF), wb),
        ],
        out_specs=pl.BlockSpec((TBX, NF), eb),
        out_shape=jax.ShapeDtypeStruct((NE, NF), F32),
    )(ns, ids_col, tabs2[2], tabs2[3], tabs2[4], tabs2[5],
      tabs2[0], tabs2[1], edge, *rw)


def kernel(atom_fea, edge, crystal_atom_idx, crystal_edge_idx, nbr_fea_idx,
           rbf, W_full, W_mask, res_W1a, res_b1a, res_W2a, res_b2a,
           res_W1b, res_b1b, res_W2b, res_b2b, gamma1, beta1, gamma2, beta2):
    ids_i32 = crystal_edge_idx.astype(jnp.int32)
    ids_2d = ids_i32.reshape(NE // GB, GB)
    ids_col = ids_i32.reshape(NE, 1)
    nbr_t = nbr_fea_idx.astype(jnp.int32).T
    idx0 = nbr_t[0].reshape(NE // GB, GB)
    idx1 = nbr_t[1].reshape(NE // GB, GB)
    w1 = W_full[:AF, :]
    w2 = W_full[AF:, :]
    wm_pad = jnp.concatenate([jnp.zeros((NF, 1), F32), W_mask], axis=0)

    proj = _tc_proj(atom_fea, w1)
    ep = _tc_ep(edge, w2)
    tg, s1, q1, cnt = _make_gather_tg_stats()(proj, idx0, idx1, ids_2d, ep)

    tabs1 = _tc_finalize(D1, s1, q1, cnt, gamma1, beta1)
    ns = _tc_gate(tg, ids_col, tabs1, wm_pad)

    s2, q2 = _make_stats(NF)(ns, ids_2d)
    tabs2 = _tc_finalize(NF, s2, q2, cnt, gamma2, beta2)

    rw = (res_W1a, res_b1a.reshape(1, -1), res_W2a, res_b2a.reshape(1, -1),
          res_W1b, res_b1b.reshape(1, -1), res_W2b, res_b2b.reshape(1, -1))
    return _tc_final(ns, ids_col, tabs2, edge, rw)


# speedup vs baseline: 2.3090x; 1.0029x over previous
import functools

import jax
import jax.numpy as jnp
from jax import lax
from jax.experimental import pallas as pl
from jax.experimental.pallas import tpu as pltpu
from jax.experimental.pallas import tpu_sc as plsc

F32 = jnp.float32
AF = 128
NF = 16
D1 = 2 * NF
NN = 10000
NE = 320000
NC = 1000
EPS = 1e-5
INV_SQRT_2 = 1.0 / 2.0 ** 0.5

GB = 128
TB = 6400
TBX = 6400
EW = 64

_HIGH = jax.lax.Precision.HIGHEST


def _mesh():
    return plsc.VectorSubcoreMesh(core_axis_name="c", subcore_axis_name="s")


_SC_PARAMS = pltpu.CompilerParams(use_tc_tiling_on_sc=False)


def _proj_body(x_ref, w_ref, o_ref):
    o_ref[...] = jnp.dot(x_ref[...], w_ref[...], precision=_HIGH)


def _tc_proj(atom_fea, w1):
    return pl.pallas_call(
        _proj_body,
        out_shape=jax.ShapeDtypeStruct((NN, D1), F32),
    )(atom_fea, w1)


def _ep_body(e_ref, w_ref, o_ref):
    o_ref[...] = jnp.dot(e_ref[...], w_ref[...])


def _tc_ep(edge, w2):
    nb = NE // TB
    return pl.pallas_call(
        _ep_body,
        grid=(nb,),
        in_specs=[
            pl.BlockSpec((TB, NF), lambda i: (i, 0)),
            pl.BlockSpec((NF, D1), lambda i: (0, 0)),
        ],
        out_specs=pl.BlockSpec((TB, D1), lambda i: (i, 0)),
        out_shape=jax.ShapeDtypeStruct((NE, D1), F32),
    )(edge, w2)


SB = 512
SUB = SB // GB


def _make_gather_tg_stats():
    @functools.partial(
        pl.kernel,
        out_type=(
            jax.ShapeDtypeStruct((NE, D1), F32),
            jax.ShapeDtypeStruct((2, NC, D1), F32),
            jax.ShapeDtypeStruct((2, NC, D1), F32),
            jax.ShapeDtypeStruct((2, NC, 16), F32),
        ),
        mesh=_mesh(),
        compiler_params=_SC_PARAMS,
        scratch_types=[
            pltpu.VMEM((SB, D1), F32),
            pltpu.VMEM((SB, D1), F32),
            pltpu.VMEM((SB, D1), F32),
            pltpu.VMEM((GB, 16), F32),
            pltpu.VMEM((125, D1), F32),
            pltpu.VMEM_SHARED((NC, D1), F32),
            pltpu.VMEM_SHARED((NC, D1), F32),
            pltpu.VMEM_SHARED((NC, 16), F32),
            pltpu.SemaphoreType.DMA,
            pltpu.SemaphoreType.DMA,
        ],
    )
    def gather_tg(proj_hbm, idx0_hbm, idx1_hbm, ids_hbm, ep_hbm,
                  tg_hbm, sum_hbm, sq_hbm, cnt_hbm,
                  p0_v, p1_v, sq_v, ones_v, z_v, ssum, ssq, scnt,
                  gsem, wsem):
        cid = lax.axis_index("c")
        sid = lax.axis_index("s")

        @pl.when(sid == 0)
        def _():
            @pl.loop(0, 125)
            def _(r):
                @pl.loop(0, D1, step=16)
                def _(c):
                    z_v.at[pl.ds(r, 1), pl.ds(c, 16)][...] = jnp.zeros(
                        (1, 16), F32)

            @pl.loop(0, 8)
            def _(k):
                pltpu.sync_copy(z_v, ssum.at[pl.ds(k * 125, 125)])
                pltpu.sync_copy(z_v, ssq.at[pl.ds(k * 125, 125)])
                pltpu.sync_copy(z_v.at[:, pl.ds(0, 16)],
                                scnt.at[pl.ds(k * 125, 125)])

        @pl.loop(0, GB)
        def _(r):
            ones_v.at[pl.ds(r, 1), pl.ds(0, 16)][...] = jnp.ones((1, 16), F32)

        plsc.subcore_barrier()

        def body(idx0_v, idx1_v, ids_v, ep_v, tg_v):
            cps = []
            for k in range(SUB):
                dst = pl.ds(k * GB, GB)
                cps.append(pltpu.async_copy(
                    proj_hbm.at[idx0_v.at[k]], p0_v.at[dst], gsem))
                cps.append(pltpu.async_copy(
                    proj_hbm.at[idx1_v.at[k]], p1_v.at[dst], gsem))
            for cp in cps:
                cp.wait()

            @pl.loop(0, SB)
            def _(r):
                @pl.loop(0, D1, step=16)
                def _(c):
                    slc = (pl.ds(r, 1), pl.ds(c, 16))
                    v = (p1_v.at[slc][...] - p0_v.at[slc][...]
                         + ep_v.at[slc][...])
                    tg_v.at[slc][...] = v
                    sq_v.at[slc][...] = v * v

            cps = []
            for k in range(SUB):
                src = pl.ds(k * GB, GB)
                cps.append(pltpu.async_copy(
                    tg_v.at[src], ssum.at[ids_v.at[k]], wsem, add=True))
                cps.append(pltpu.async_copy(
                    sq_v.at[src], ssq.at[ids_v.at[k]], wsem, add=True))
                cps.append(pltpu.async_copy(
                    ones_v, scnt.at[ids_v.at[k]], wsem, add=True))
            for cp in cps:
                cp.wait()

        pltpu.emit_pipeline(
            body,
            grid=(NE // SB,),
            in_specs=[
                pl.BlockSpec((SUB, GB), lambda i: (i, 0)),
                pl.BlockSpec((SUB, GB), lambda i: (i, 0)),
                pl.BlockSpec((SUB, GB), lambda i: (i, 0)),
                pl.BlockSpec((SB, D1), lambda i: (i, 0)),
            ],
            out_specs=[pl.BlockSpec((SB, D1), lambda i: (i, 0))],
            core_axis_name=("c", "s"),
            dimension_semantics=(pltpu.PARALLEL,),
        )(idx0_hbm, idx1_hbm, ids_hbm, ep_hbm, tg_hbm)

        plsc.subcore_barrier()

        @pl.when(sid == 0)
        def _():
            pltpu.sync_copy(ssum, sum_hbm.at[cid])
            pltpu.sync_copy(ssq, sq_hbm.at[cid])
            pltpu.sync_copy(scnt, cnt_hbm.at[cid])

    return gather_tg


def _make_stats(d):
    @functools.partial(
        pl.kernel,
        out_type=(
            jax.ShapeDtypeStruct((2, NC, d), F32),
            jax.ShapeDtypeStruct((2, NC, d), F32),
        ),
        mesh=_mesh(),
        compiler_params=_SC_PARAMS,
        scratch_types=[
            pltpu.VMEM((SB, d), F32),
            pltpu.VMEM((125, d), F32),
            pltpu.VMEM_SHARED((NC, d), F32),
            pltpu.VMEM_SHARED((NC, d), F32),
            pltpu.SemaphoreType.DMA,
        ],
    )
    def stats(x_hbm, ids_hbm, sum_hbm, sq_hbm, sq_v, z_v, ssum, ssq, wsem):
        cid = lax.axis_index("c")
        sid = lax.axis_index("s")

        @pl.when(sid == 0)
        def _():
            @pl.loop(0, 125)
            def _(r):
                @pl.loop(0, d, step=16)
                def _(c):
                    z_v.at[pl.ds(r, 1), pl.ds(c, 16)][...] = jnp.zeros(
                        (1, 16), F32)

            @pl.loop(0, 8)
            def _(k):
                pltpu.sync_copy(z_v, ssum.at[pl.ds(k * 125, 125)])
                pltpu.sync_copy(z_v, ssq.at[pl.ds(k * 125, 125)])

        plsc.subcore_barrier()

        def body(x_v, ids_v):
            @pl.loop(0, SB)
            def _(r):
                @pl.loop(0, d, step=16)
                def _(c):
                    slc = (pl.ds(r, 1), pl.ds(c, 16))
                    v = x_v.at[slc][...]
                    sq_v.at[slc][...] = v * v

            cps = []
            for k in range(SUB):
                src = pl.ds(k * GB, GB)
                cps.append(pltpu.async_copy(
                    x_v.at[src], ssum.at[ids_v.at[k]], wsem, add=True))
                cps.append(pltpu.async_copy(
                    sq_v.at[src], ssq.at[ids_v.at[k]], wsem, add=True))
            for cp in cps:
                cp.wait()

        pltpu.emit_pipeline(
            body,
            grid=(NE // SB,),
            in_specs=[
                pl.BlockSpec((SB, d), lambda i: (i, 0)),
                pl.BlockSpec((SUB, GB), lambda i: (i, 0)),
            ],
            out_specs=[],
            core_axis_name=("c", "s"),
            dimension_semantics=(pltpu.PARALLEL,),
        )(x_hbm, ids_hbm)

        plsc.subcore_barrier()

        @pl.when(sid == 0)
        def _():
            pltpu.sync_copy(ssum, sum_hbm.at[cid])
            pltpu.sync_copy(ssq, sq_hbm.at[cid])

    return stats


def _fin_body(sum_ref, sq_ref, cnt_ref, g_ref, bt_ref,
              a_ref, b_ref, ah_ref, al_ref, bh_ref, bl_ref):
    s = sum_ref[0] + sum_ref[1]
    q = sq_ref[0] + sq_ref[1]
    n = jnp.maximum(cnt_ref[0, :, 0:1] + cnt_ref[1, :, 0:1], 1.0)
    mean = s / n
    var = jnp.maximum(q / n - mean * mean, 0.0)
    a = g_ref[...] * lax.rsqrt(var + EPS)
    b = bt_ref[...] - mean * a
    a_ref[...] = a
    b_ref[...] = b
    ah = a.astype(jnp.bfloat16)
    bh = b.astype(jnp.bfloat16)
    ah_ref[...] = ah
    bh_ref[...] = bh
    al_ref[...] = (a - ah.astype(F32)).astype(jnp.bfloat16)
    bl_ref[...] = (b - bh.astype(F32)).astype(jnp.bfloat16)


def _tc_finalize(d, ssum, ssq, cnt, gamma, beta):
    bf = jnp.bfloat16
    return pl.pallas_call(
        _fin_body,
        out_shape=(
            jax.ShapeDtypeStruct((NC, d), F32),
            jax.ShapeDtypeStruct((NC, d), F32),
            jax.ShapeDtypeStruct((NC, d), bf),
            jax.ShapeDtypeStruct((NC, d), bf),
            jax.ShapeDtypeStruct((NC, d), bf),
            jax.ShapeDtypeStruct((NC, d), bf),
        ),
    )(ssum, ssq, cnt, gamma.reshape(1, d), beta.reshape(1, d))


def _expand_tables(ids, ah_ref, al_ref, bh_ref, bl_ref, a_ref, b_ref, d):
    n = ids.shape[0]
    clo = jnp.min(ids)
    chi = jnp.max(ids)
    base = pl.multiple_of(jnp.minimum((clo // 8) * 8, NC - EW), 8)
    lid = ids - base
    iota = lax.broadcasted_iota(jnp.int32, (1, EW), 1)
    oh = (lid == iota).astype(jnp.bfloat16)
    wah = ah_ref[pl.ds(base, EW), :]
    wal = al_ref[pl.ds(base, EW), :]
    wbh = bh_ref[pl.ds(base, EW), :]
    wbl = bl_ref[pl.ds(base, EW), :]
    a_pe = (jnp.dot(oh, wah, preferred_element_type=F32)
            + jnp.dot(oh, wal, preferred_element_type=F32))
    b_pe = (jnp.dot(oh, wbh, preferred_element_type=F32)
            + jnp.dot(oh, wbl, preferred_element_type=F32))

    def it(c, ab):
        a_acc, b_acc = ab
        m = (ids == c).astype(F32)
        a_acc = a_acc + m * a_ref[pl.ds(c, 1), :]
        b_acc = b_acc + m * b_ref[pl.ds(c, 1), :]
        return (a_acc, b_acc)

    return lax.fori_loop(base + EW, chi + 1, it, (a_pe, b_pe))


def _gate_body(tg_ref, ids_ref, ah_ref, al_ref, bh_ref, bl_ref,
               a_ref, b_ref, wm_ref, ns_ref):
    ids = ids_ref[...]
    a_pe, b_pe = _expand_tables(ids, ah_ref, al_ref, bh_ref, bl_ref,
                                a_ref, b_ref, D1)
    tgn = tg_ref[...] * a_pe + b_pe
    filt = jnp.tanh(jnp.dot(tgn, wm_ref[...]))
    ns_ref[...] = (jax.nn.relu(tgn) * filt)[:, :NF]


def _tc_gate(tg, ids_col, tabs1, wm_pad):
    nb = NE // TBX
    eb = lambda i: (i, 0)
    wb = lambda i: (0, 0)
    return pl.pallas_call(
        _gate_body,
        grid=(nb,),
        in_specs=[
            pl.BlockSpec((TBX, D1), eb),
            pl.BlockSpec((TBX, 1), eb),
            pl.BlockSpec((NC, D1), wb),
            pl.BlockSpec((NC, D1), wb),
            pl.BlockSpec((NC, D1), wb),
            pl.BlockSpec((NC, D1), wb),
            pl.BlockSpec((NC, D1), wb),
            pl.BlockSpec((NC, D1), wb),
            pl.BlockSpec((D1, 1), wb),
        ],
        out_specs=pl.BlockSpec((TBX, NF), eb),
        out_shape=jax.ShapeDtypeStruct((NE, NF), F32),
    )(tg, ids_col, tabs1[2], tabs1[3], tabs1[4], tabs1[5],
      tabs1[0], tabs1[1], wm_pad)


def _final_body(ns_ref, ids_ref, ah_ref, al_ref, bh_ref, bl_ref,
                a_ref, b_ref, e_ref,
                w1a_ref, b1a_ref, w2a_ref, b2a_ref,
                w1b_ref, b1b_ref, w2b_ref, b2b_ref, o_ref):
    ids = ids_ref[...]
    a_pe, b_pe = _expand_tables(ids, ah_ref, al_ref, bh_ref, bl_ref,
                                a_ref, b_ref, NF)
    x = ns_ref[...] * a_pe + b_pe
    h = jnp.dot(jax.nn.relu(jnp.dot(x, w1a_ref[...]) + b1a_ref[...]),
                w2a_ref[...]) + b2a_ref[...]
    x = x + h
    h = jnp.dot(jax.nn.relu(jnp.dot(x, w1b_ref[...]) + b1b_ref[...]),
                w2b_ref[...]) + b2b_ref[...]
    x = x + h
    o_ref[...] = INV_SQRT_2 * jax.nn.relu(e_ref[...] + x)


def _tc_final(ns, ids_col, tabs2, edge, rw):
    nb = NE // TBX
    mid = NF // 2
    eb = lambda i: (i, 0)
    wb = lambda i: (0, 0)
    return pl.pallas_call(
        _final_body,
        grid=(nb,),
        in_specs=[
            pl.BlockSpec((TBX, NF), eb),
            pl.BlockSpec((TBX, 1), eb),
            pl.BlockSpec((NC, NF), wb),
            pl.BlockSpec((NC, NF), wb),
            pl.BlockSpec((NC, NF), wb),
            pl.BlockSpec((NC, NF), wb),
            pl.BlockSpec((NC, NF), wb),
            pl.BlockSpec((NC, NF), wb),
            pl.BlockSpec((TBX, NF), eb),
            pl.BlockSpec((NF, mid), wb),
            pl.BlockSpec((1, mid), wb),
            pl.BlockSpec((mid, NF), wb),
            pl.BlockSpec((1, NF), wb),
            pl.BlockSpec((NF, mid), wb),
            pl.BlockSpec((1, mid), wb),
            pl.BlockSpec((mid, NF), wb),
            pl.BlockSpec((1, NF), wb),
        ],
        out_specs=pl.BlockSpec((TBX, NF), eb),
        out_shape=jax.ShapeDtypeStruct((NE, NF), F32),
    )(ns, ids_col, tabs2[2], tabs2[3], tabs2[4], tabs2[5],
      tabs2[0], tabs2[1], edge, *rw)


def kernel(atom_fea, edge, crystal_atom_idx, crystal_edge_idx, nbr_fea_idx,
           rbf, W_full, W_mask, res_W1a, res_b1a, res_W2a, res_b2a,
           res_W1b, res_b1b, res_W2b, res_b2b, gamma1, beta1, gamma2, beta2):
    ids_i32 = crystal_edge_idx.astype(jnp.int32)
    ids_2d = ids_i32.reshape(NE // GB, GB)
    ids_col = ids_i32.reshape(NE, 1)
    nbr_t = nbr_fea_idx.astype(jnp.int32).T
    idx0 = nbr_t[0].reshape(NE // GB, GB)
    idx1 = nbr_t[1].reshape(NE // GB, GB)
    w1 = W_full[:AF, :]
    w2 = W_full[AF:, :]
    wm_pad = jnp.concatenate([jnp.zeros((NF, 1), F32), W_mask], axis=0)

    proj = _tc_proj(atom_fea, w1)
    ep = _tc_ep(edge, w2)
    tg, s1, q1, cnt = _make_gather_tg_stats()(proj, idx0, idx1, ids_2d, ep)

    tabs1 = _tc_finalize(D1, s1, q1, cnt, gamma1, beta1)
    ns = _tc_gate(tg, ids_col, tabs1, wm_pad)

    s2, q2 = _make_stats(NF)(ns, ids_2d)
    tabs2 = _tc_finalize(NF, s2, q2, cnt, gamma2, beta2)

    rw = (res_W1a, res_b1a.reshape(1, -1), res_W2a, res_b2a.reshape(1, -1),
          res_W1b, res_b1b.reshape(1, -1), res_W2b, res_b2b.reshape(1, -1))
    return _tc_final(ns, ids_col, tabs2, edge, rw)
